# baseline (speedup 1.0000x reference)
import jax
import jax.numpy as jnp
from jax import lax
from jax.experimental import pallas as pl
from jax.experimental.pallas import tpu as pltpu
from jax.experimental.pallas import tpu_sc as plsc

D = 512
CAND = 128
BATCH = 16
LANES = 16
ROWS_PER_W = 64
CHUNKS = D // LANES
HALF = ROWS_PER_W // 2


def _sc_body(offs_hbm, nco_hbm, values_hbm,
             out_hbm, nseq_hbm, noffs_hbm,
             offs_v, nco_v, nseq_v, rows_v, out_v,
             sem_a, sem_b, sem_o):
    cid = lax.axis_index("c")
    sid = lax.axis_index("s")
    wid = sid * 2 + cid
    b = wid // 2
    half = wid - 2 * b

    pltpu.sync_copy(offs_hbm, offs_v)
    ends = offs_v[pl.ds(1, LANES)]
    lane = lax.iota(jnp.int32, 16)
    end_b = jnp.sum(jnp.where(lane == b, ends, 0))
    start = end_b - CAND + half * ROWS_PER_W

    start_al = (start // 8) * 8
    phase = start - start_al
    cp_a = pltpu.make_async_copy(
        values_hbm.at[pl.ds(start_al, HALF + 8)],
        rows_v.at[pl.ds(0, HALF + 8)], sem_a)
    cp_b = pltpu.make_async_copy(
        values_hbm.at[pl.ds(start_al + HALF + 8, HALF)],
        rows_v.at[pl.ds(HALF + 8, HALF)], sem_b)
    cp_a.start()
    cp_b.start()

    @pl.when(wid == 0)
    def _aux():
        pltpu.sync_copy(nco_hbm, nco_v)
        nseq_v[...] = nco_v[pl.ds(1, LANES)] - nco_v[pl.ds(0, LANES)]
        pltpu.sync_copy(nseq_v, nseq_hbm)
        pltpu.sync_copy(nco_hbm, noffs_hbm)

    def _norm_half(base):
        @plsc.parallel_loop(base, base + HALF, step=1)
        def _row(r):
            rs = r + phase
            xs = []
            acc = jnp.zeros((LANES,), jnp.float32)
            for c in range(CHUNKS):
                x = rows_v[rs, pl.ds(c * LANES, LANES)]
                xs.append(x)
                acc = acc + x * x
            s = jnp.sum(acc)
            sv = jnp.zeros((LANES,), jnp.float32) + s
            iv = plsc.bitcast(sv, jnp.int32)
            y = plsc.bitcast(jnp.int32(0x5F3759DF) - (iv >> 1), jnp.float32)
            for _ in range(3):
                y = y * (1.5 - 0.5 * sv * y * y)
            norm = sv * y
            inv = jnp.where(norm > 1e-6, y, 1e6)
            for c in range(CHUNKS):
                out_v[r, pl.ds(c * LANES, LANES)] = xs[c] * inv

    cp_a.wait()
    _norm_half(0)
    cp_o = pltpu.make_async_copy(
        out_v.at[pl.ds(0, HALF)],
        out_hbm.at[pl.ds(wid * ROWS_PER_W, HALF)], sem_o)
    cp_o.start()
    cp_b.wait()
    _norm_half(HALF)
    cp_o.wait()
    pltpu.sync_copy(out_v.at[pl.ds(HALF, HALF)],
                    out_hbm.at[pl.ds(wid * ROWS_PER_W + HALF, HALF)])


@jax.jit
def _sc_split_norm(offs, nco, values):
    mesh = plsc.VectorSubcoreMesh(core_axis_name="c", subcore_axis_name="s")
    return pl.kernel(
        _sc_body,
        out_type=[
            jax.ShapeDtypeStruct((BATCH * CAND, D), jnp.float32),
            jax.ShapeDtypeStruct((BATCH,), jnp.int32),
            jax.ShapeDtypeStruct((BATCH + 1,), jnp.int32),
        ],
        mesh=mesh,
        scratch_types=[
            pltpu.VMEM((BATCH + 1,), jnp.int32),
            pltpu.VMEM((BATCH + 1,), jnp.int32),
            pltpu.VMEM((BATCH,), jnp.int32),
            pltpu.VMEM((ROWS_PER_W + 8, D), jnp.float32),
            pltpu.VMEM((ROWS_PER_W, D), jnp.float32),
            pltpu.SemaphoreType.DMA,
            pltpu.SemaphoreType.DMA,
            pltpu.SemaphoreType.DMA,
        ],
        compiler_params=pltpu.CompilerParams(needs_layout_passes=False),
    )(offs, nco, values)


def kernel(values, seqlen_offsets, num_candidates_offsets, max_seqlen, max_num_candidates):
    offs = seqlen_offsets.astype(jnp.int32)
    nco = num_candidates_offsets.astype(jnp.int32)
    return _sc_split_norm(offs, nco, values)


# speedup vs baseline: 1.0074x; 1.0074x over previous
import jax
import jax.numpy as jnp
from jax import lax
from jax.experimental import pallas as pl
from jax.experimental.pallas import tpu as pltpu
from jax.experimental.pallas import tpu_sc as plsc

D = 512
CAND = 128
BATCH = 16
LANES = 16
ROWS_PER_W = 64
CHUNKS = D // LANES
HALF = ROWS_PER_W // 2


def _sc_body(offs_hbm, nco_hbm, values_hbm,
             out_hbm, nseq_hbm, noffs_hbm,
             offs_v, nco_v, nseq_v, rows_v, out_v,
             sem_a, sem_b, sem_o):
    cid = lax.axis_index("c")
    sid = lax.axis_index("s")
    wid = sid * 2 + cid
    b = wid // 2
    half = wid - 2 * b

    pltpu.sync_copy(offs_hbm, offs_v)
    ends = offs_v[pl.ds(1, LANES)]
    lane = lax.iota(jnp.int32, 16)
    end_b = jnp.sum(jnp.where(lane == b, ends, 0))
    start = end_b - CAND + half * ROWS_PER_W

    start_al = (start // 8) * 8
    phase = start - start_al
    cp_a = pltpu.make_async_copy(
        values_hbm.at[pl.ds(start_al, HALF + 8)],
        rows_v.at[pl.ds(0, HALF + 8)], sem_a)
    cp_b = pltpu.make_async_copy(
        values_hbm.at[pl.ds(start_al + HALF + 8, HALF)],
        rows_v.at[pl.ds(HALF + 8, HALF)], sem_b)
    cp_a.start()
    cp_b.start()

    @pl.when(wid == 0)
    def _aux():
        pltpu.sync_copy(nco_hbm, nco_v)
        nseq_v[...] = nco_v[pl.ds(1, LANES)] - nco_v[pl.ds(0, LANES)]
        pltpu.sync_copy(nseq_v, nseq_hbm)
        pltpu.sync_copy(nco_hbm, noffs_hbm)

    def _norm_half(base):
        @plsc.parallel_loop(base, base + HALF, step=1)
        def _row(r):
            rs = r + phase
            xs = []
            acc = jnp.zeros((LANES,), jnp.float32)
            for c in range(CHUNKS):
                x = rows_v[rs, pl.ds(c * LANES, LANES)]
                xs.append(x)
                acc = acc + x * x
            s = jnp.sum(acc)
            sv = jnp.zeros((LANES,), jnp.float32) + s
            iv = plsc.bitcast(sv, jnp.int32)
            y = plsc.bitcast(jnp.int32(0x5F3759DF) - (iv >> 1), jnp.float32)
            for _ in range(3):
                y = y * (1.5 - 0.5 * sv * y * y)
            norm = sv * y
            inv = jnp.where(norm > 1e-6, y, 1e6)
            for c in range(CHUNKS):
                out_v[r, pl.ds(c * LANES, LANES)] = xs[c] * inv

    cp_a.wait()
    _norm_half(0)
    cp_o = pltpu.make_async_copy(
        out_v.at[pl.ds(0, HALF)],
        out_hbm.at[pl.ds(wid * ROWS_PER_W, HALF)], sem_o)
    cp_o.start()
    cp_b.wait()
    _norm_half(HALF)
    cp_o.wait()
    pltpu.sync_copy(out_v.at[pl.ds(HALF, HALF)],
                    out_hbm.at[pl.ds(wid * ROWS_PER_W + HALF, HALF)])


@jax.jit
def _sc_split_norm(offs, nco, values):
    mesh = plsc.VectorSubcoreMesh(core_axis_name="c", subcore_axis_name="s")
    return pl.kernel(
        _sc_body,
        out_type=[
            jax.ShapeDtypeStruct((BATCH * CAND, D), jnp.float32),
            jax.ShapeDtypeStruct((BATCH,), jnp.int32),
            jax.ShapeDtypeStruct((BATCH + 1,), jnp.int32),
        ],
        mesh=mesh,
        scratch_types=[
            pltpu.VMEM((BATCH + 1,), jnp.int32),
            pltpu.VMEM((BATCH + 1,), jnp.int32),
            pltpu.VMEM((BATCH,), jnp.int32),
            pltpu.VMEM((ROWS_PER_W + 8, D), jnp.float32),
            pltpu.VMEM((ROWS_PER_W, D), jnp.float32),
            pltpu.SemaphoreType.DMA,
            pltpu.SemaphoreType.DMA,
            pltpu.SemaphoreType.DMA,
        ],
        compiler_params=pltpu.CompilerParams(needs_layout_passes=False),
    )(offs, nco, values)


def kernel(values, seqlen_offsets, num_candidates_offsets, max_seqlen, max_num_candidates):
    offs = seqlen_offsets.astype(jnp.int32)
    nco = num_candidates_offsets.astype(jnp.int32)
    emb, new_seqlen, new_offsets = _sc_split_norm(offs, nco, values)
    return emb, new_seqlen, new_offsets


# speedup vs baseline: 1.0239x; 1.0164x over previous
import jax
import jax.numpy as jnp
from jax import lax
from jax.experimental import pallas as pl
from jax.experimental.pallas import tpu as pltpu
from jax.experimental.pallas import tpu_sc as plsc

D = 512
CAND = 128
BATCH = 16
LANES = 16
ROWS_PER_W = 64
CHUNKS = D // LANES
HALF = ROWS_PER_W // 2


def _sc_body(offs_hbm, nco_hbm, values_hbm,
             out_hbm, nseq_hbm, noffs_hbm,
             offs_v, nco_v, nseq_v, rows_v, out_v,
             sem_a, sem_b, sem_o):
    cid = lax.axis_index("c")
    sid = lax.axis_index("s")
    wid = sid * 2 + cid
    b = wid // 2
    half = wid - 2 * b

    pltpu.sync_copy(offs_hbm, offs_v)
    ends = offs_v[pl.ds(1, LANES)]
    lane = lax.iota(jnp.int32, 16)
    end_b = jnp.sum(jnp.where(lane == b, ends, 0))
    start = end_b - CAND + half * ROWS_PER_W

    start_al = (start // 8) * 8
    phase = start - start_al
    cp_in = pltpu.make_async_copy(
        values_hbm.at[pl.ds(start_al, ROWS_PER_W + 8)], rows_v, sem_a)
    cp_in.start()

    @pl.when(wid == 0)
    def _aux():
        pltpu.sync_copy(nco_hbm, nco_v)
        nseq_v[...] = nco_v[pl.ds(1, LANES)] - nco_v[pl.ds(0, LANES)]
        pltpu.sync_copy(nseq_v, nseq_hbm)
        pltpu.sync_copy(nco_hbm, noffs_hbm)

    def _norm_half(base, nrows):
        @plsc.parallel_loop(base, base + nrows, step=1)
        def _row(r):
            rs = r + phase
            xs = []
            acc = jnp.zeros((LANES,), jnp.float32)
            for c in range(CHUNKS):
                x = rows_v[rs, pl.ds(c * LANES, LANES)]
                xs.append(x)
                acc = acc + x * x
            s = jnp.sum(acc)
            sv = jnp.zeros((LANES,), jnp.float32) + s
            iv = plsc.bitcast(sv, jnp.int32)
            y = plsc.bitcast(jnp.int32(0x5F3759DF) - (iv >> 1), jnp.float32)
            for _ in range(3):
                y = y * (1.5 - 0.5 * sv * y * y)
            norm = sv * y
            inv = jnp.where(norm > 1e-6, y, 1e6)
            for c in range(CHUNKS):
                out_v[r, pl.ds(c * LANES, LANES)] = xs[c] * inv

    cp_in.wait()
    _norm_half(0, ROWS_PER_W)
    pltpu.sync_copy(out_v, out_hbm.at[pl.ds(wid * ROWS_PER_W, ROWS_PER_W)])


@jax.jit
def _sc_split_norm(offs, nco, values):
    mesh = plsc.VectorSubcoreMesh(core_axis_name="c", subcore_axis_name="s")
    return pl.kernel(
        _sc_body,
        out_type=[
            jax.ShapeDtypeStruct((BATCH * CAND, D), jnp.float32),
            jax.ShapeDtypeStruct((BATCH,), jnp.int32),
            jax.ShapeDtypeStruct((BATCH + 1,), jnp.int32),
        ],
        mesh=mesh,
        scratch_types=[
            pltpu.VMEM((BATCH + 1,), jnp.int32),
            pltpu.VMEM((BATCH + 1,), jnp.int32),
            pltpu.VMEM((BATCH,), jnp.int32),
            pltpu.VMEM((ROWS_PER_W + 8, D), jnp.float32),
            pltpu.VMEM((ROWS_PER_W, D), jnp.float32),
            pltpu.SemaphoreType.DMA,
            pltpu.SemaphoreType.DMA,
            pltpu.SemaphoreType.DMA,
        ],
        compiler_params=pltpu.CompilerParams(needs_layout_passes=False),
    )(offs, nco, values)


def kernel(values, seqlen_offsets, num_candidates_offsets, max_seqlen, max_num_candidates):
    offs = seqlen_offsets.astype(jnp.int32)
    nco = num_candidates_offsets.astype(jnp.int32)
    emb, new_seqlen, new_offsets = _sc_split_norm(offs, nco, values)
    return emb, new_seqlen, new_offsets
